# Initial kernel scaffold; baseline (speedup 1.0000x reference)
import jax
import jax.numpy as jnp
from jax.experimental import pallas as pl


def kernel(pcl, prev_pcl, weight_logits, nn_idxs):
    raise NotImplementedError("write your pallas kernel here")


# speedup vs baseline: 33.6010x; 33.6010x over previous
import functools

import jax
import jax.numpy as jnp
from jax import lax
from jax.experimental import pallas as pl
from jax.experimental.pallas import tpu as pltpu
from jax.experimental.pallas import tpu_sc as plsc

_NC, _NS, _L = 2, 16, 16
_NW = _NC * _NS


def _diff_body(a_ref, b_ref, o_ref):
    o_ref[...] = a_ref[...] - b_ref[...]


def _sc_body(npad, chunk, k_nn, diff_hbm, idx_hbm, lg_hbm, out_hbm,
             table_v, idx_v, w_v, acc_v):
    ngroups = chunk // _L
    wid = lax.axis_index("s") * _NC + lax.axis_index("c")
    base = wid * chunk
    kc = chunk * k_nn

    pltpu.sync_copy(idx_hbm.at[pl.ds(wid * kc, kc)], idx_v)
    pltpu.sync_copy(lg_hbm.at[pl.ds(wid * kc, kc)], w_v)

    def smax_body(j, carry):
        off = j * _L
        ls = [w_v[pl.ds(k * chunk + off, _L)] for k in range(k_nn)]
        m = ls[0]
        for l in ls[1:]:
            m = jnp.maximum(m, l)
        es = [jnp.exp(l - m) for l in ls]
        s = es[0]
        for e in es[1:]:
            s = s + e
        inv = 1.0 / s
        for k in range(k_nn):
            w_v[pl.ds(k * chunk + off, _L)] = es[k] * inv
        return carry
    lax.fori_loop(0, ngroups, smax_body, 0)

    total = jnp.zeros((_L,), jnp.float32)
    for c in range(3):
        pltpu.sync_copy(diff_hbm.at[pl.ds(c * npad, npad)], table_v)

        def group_body(j, acc):
            off = j * _L
            r = table_v[pl.ds(base + off, _L)]
            for k in range(k_nn):
                ik = idx_v[pl.ds(k * chunk + off, _L)]
                wk = w_v[pl.ds(k * chunk + off, _L)]
                g = plsc.load_gather(table_v, [ik])
                r = r - wk * g
            return acc + r * r
        total = lax.fori_loop(0, ngroups, group_body, total)

    acc_v[...] = total
    pltpu.sync_copy(acc_v, out_hbm.at[pl.ds(wid * _L, _L)])


def kernel(pcl, prev_pcl, weight_logits, nn_idxs):
    n = pcl.shape[0]
    k_nn = nn_idxs.shape[1]
    npad = -(-n // (_NW * _L)) * (_NW * _L)
    chunk = npad // _NW
    pad = npad - n

    pclT = jnp.pad(pcl, ((0, pad), (0, 0))).T
    prevT = jnp.pad(prev_pcl, ((0, pad), (0, 0))).T
    diffT = pl.pallas_call(
        _diff_body,
        out_shape=jax.ShapeDtypeStruct((3, npad), jnp.float32),
    )(pclT, prevT)

    idxB = jnp.pad(nn_idxs.astype(jnp.int32), ((0, pad), (0, 0)),
                   constant_values=n)
    idxB = idxB.reshape(_NW, chunk, k_nn).transpose(0, 2, 1).reshape(-1)
    lgB = jnp.pad(weight_logits, ((0, pad), (0, 0)))
    lgB = lgB.reshape(_NW, chunk, k_nn).transpose(0, 2, 1).reshape(-1)

    mesh = plsc.VectorSubcoreMesh(core_axis_name="c", subcore_axis_name="s")
    partials = pl.kernel(
        functools.partial(_sc_body, npad, chunk, k_nn),
        out_type=jax.ShapeDtypeStruct((_NW * _L,), jnp.float32),
        mesh=mesh,
        compiler_params=pltpu.CompilerParams(
            needs_layout_passes=False, use_tc_tiling_on_sc=False),
        scratch_types=[
            pltpu.VMEM((npad,), jnp.float32),
            pltpu.VMEM((k_nn * chunk,), jnp.int32),
            pltpu.VMEM((k_nn * chunk,), jnp.float32),
            pltpu.VMEM((_L,), jnp.float32),
        ],
    )(diffT.reshape(-1), idxB, lgB)

    return jnp.sum(partials) / (3.0 * n)
